# baseline (speedup 1.0000x reference)
import jax
import jax.numpy as jnp
from jax.experimental import pallas as pl
from jax.experimental.pallas import tpu as pltpu

BATCH = 8
T = 32
D = 512
VOCAB = 4096
BEAM = 5
BB = BATCH * BEAM
MAX_SYM = 3
BLANK = 0
HYP_W = 128
NEG_BIG = -1.0e30
NEG_SENT = -3.0e38

F32 = jnp.float32
HI = jax.lax.Precision.HIGHEST


def _iota(shape, dim):
    return jax.lax.broadcasted_iota(jnp.int32, shape, dim).astype(F32)


def _dot(a, b):
    return jax.lax.dot(a, b, precision=HI, preferred_element_type=F32)


def _ddot(a, b):
    return jax.lax.dot(a, b, preferred_element_type=F32)


def _topk5_iter(work, iot):
    vals, idxs = [], []
    for _ in range(5):
        m = jnp.max(work, axis=1, keepdims=True)
        hit = work == m
        ix = jnp.min(jnp.where(hit, iot, 1.0e9), axis=1, keepdims=True)
        vals.append(m)
        idxs.append(ix)
        work = jnp.where(iot == ix, NEG_SENT, work)
    return jnp.concatenate(vals, axis=1), jnp.concatenate(idxs, axis=1)


def _roll1(x, s):
    return jnp.concatenate([x[:, s:], x[:, :s]], axis=1)


def _topk5_rank(v, ix):
    n = v.shape[1]
    rank = jnp.zeros_like(v)
    for s in range(1, n):
        vr = _roll1(v, s)
        ir = _roll1(ix, s)
        rank = rank + ((vr > v) | ((vr == v) & (ir < ix))).astype(F32)
    vals, idxs = [], []
    for k in range(5):
        hit = rank == float(k)
        vals.append(jnp.sum(jnp.where(hit, v, 0.0), axis=1, keepdims=True))
        idxs.append(jnp.sum(jnp.where(hit, ix, 0.0), axis=1, keepdims=True))
    return jnp.concatenate(vals, axis=1), jnp.concatenate(idxs, axis=1)


def _topk5_wide(work, iot, chunks):
    cw = work.shape[1] // chunks
    cvs, cis = [], []
    for c in range(chunks):
        cv_, ci_ = _topk5_iter(work[:, c * cw:(c + 1) * cw],
                               iot[:, c * cw:(c + 1) * cw])
        cvs.append(cv_)
        cis.append(ci_)
    return _topk5_rank(jnp.concatenate(cvs, axis=1),
                       jnp.concatenate(cis, axis=1))


def _body(enc_ref, wej_ref, t123_ref, wdec_ref, udec_ref, bdec_ref,
          wdj_ref, bj_ref, wout_ref, bout_ref, len_ref, sinit_ref,
          out_h_ref, out_s_ref, encp_ref, st_ref, hyp_ref):
    ri40_8 = _iota((BB, BATCH), 0)
    bi40_8 = _iota((BB, BATCH), 1)
    rep = ((ri40_8 >= 5.0 * bi40_8) & (ri40_8 < 5.0 * bi40_8 + 5.0)).astype(F32)
    gkT = [(ri40_8 == 5.0 * bi40_8 + k).astype(F32) for k in range(BEAM)]
    bi8_40 = _iota((BATCH, BB), 0)
    ri8_40 = _iota((BATCH, BB), 1)
    gp = [(ri8_40 == 5.0 * bi8_40 + p).astype(F32) for p in range(BEAM)]
    base540 = _dot(rep, 5.0 * _iota((BATCH, 1), 0))
    lane_h = _iota((BB, HYP_W), 1)
    iota_v = _iota((BB, VOCAB), 1)
    iota25 = _iota((BATCH, BEAM * BEAM), 1)
    iota5_40 = _iota((BB, BEAM), 1)
    iota5_8 = _iota((BATCH, BEAM), 1)
    iota40_40 = _iota((BB, BB), 1)
    iota40_80 = _iota((BB, 2 * BB), 1)
    pi5_25 = _iota((BEAM, BEAM * BEAM), 0)
    ci5_25 = _iota((BEAM, BEAM * BEAM), 1)
    r5 = ((ci5_25 >= 5.0 * pi5_25) &
          (ci5_25 < 5.0 * pi5_25 + 5.0)).astype(F32)

    def cv(x):
        acc = jnp.zeros((BB, 1), F32)
        for k in range(BEAM):
            acc = acc + _dot(gkT[k], x[:, k:k + 1])
        return acc

    encp_ref[...] = _ddot(enc_ref[...], wej_ref[...])
    st_ref[...] = jnp.zeros((BB, D), F32)
    hyp_ref[...] = jnp.zeros((BB, HYP_W), F32)
    bdec = bdec_ref[...]
    bj = bj_ref[...]
    bout = bout_ref[...]
    enc_len = len_ref[...]

    def step_i(i, carry):
        cur_tok, last_tok, scores = carry
        i_f = jax.lax.convert_element_type(i, F32)
        enc8 = encp_ref[pl.ds(i * BATCH, BATCH), :]
        enc40p = _dot(rep, enc8)
        end_flag = jnp.zeros((BB, 1), F32)
        for j in range(MAX_SYM):
            flag = jnp.maximum(end_flag, (enc_len <= i_f).astype(F32))

            def live_path(tok):
                ohb = (iota_v == tok).astype(jnp.bfloat16)
                e123 = jax.lax.dot(ohb, t123_ref[...],
                                   preferred_element_type=F32)
                emb = ((e123[:, :D] + e123[:, D:2 * D]) + e123[:, 2 * D:])
                ns = jnp.tanh(_ddot(emb, wdec_ref[...]) +
                              _ddot(st_ref[...], udec_ref[...]) + bdec)
                h = jnp.tanh(enc40p + _ddot(ns, wdj_ref[...]) + bj)
                logits = _ddot(h, wout_ref[...]) + bout
                mx = jnp.max(logits, axis=1, keepdims=True)
                shifted = logits - mx
                ls = shifted - jnp.log(jnp.sum(jnp.exp(shifted), axis=1,
                                               keepdims=True))
                tv, ti = _topk5_wide(ls, iota_v, chunks=8)
                return tv, ti, ns

            def dead_path(tok):
                zero5 = jnp.zeros((BB, BEAM), F32)
                return zero5, zero5, jnp.zeros((BB, D), F32)

            tv, ti, ns = jax.lax.cond(jnp.min(flag) >= 1.0,
                                      dead_path, live_path, cur_tok)
            tv = jnp.where((iota5_40 > 0) & (flag > 0), NEG_BIG, tv)
            tv = jnp.where((iota5_40 == 0) & (flag > 0), 0.0, tv)
            ti = jnp.where(flag > 0, float(BLANK), ti)
            tvb = jnp.concatenate([_dot(gp[p], tv) for p in range(BEAM)],
                                  axis=1)
            a = _dot(scores, r5) + tvb
            tsv, tsi = _topk5_rank(a, iota25)
            scores = tsv
            tsi40 = cv(tsi)
            p40 = ((tsi40 >= 5.0).astype(F32) + (tsi40 >= 10.0).astype(F32) +
                   (tsi40 >= 15.0).astype(F32) + (tsi40 >= 20.0).astype(F32))
            q40 = tsi40 - 5.0 * p40
            bh40 = base540 + p40
            perm = (iota40_40 == bh40).astype(F32)
            pt = _dot(perm, jnp.concatenate([hyp_ref[...], ti, last_tok],
                                            axis=1))
            phyp = pt[:, :HYP_W]
            ptop = pt[:, HYP_W:HYP_W + BEAM]
            plast = pt[:, HYP_W + BEAM:HYP_W + BEAM + 1]
            ntok = jnp.sum(ptop * (iota5_40 == q40).astype(F32), axis=1,
                           keepdims=True)
            end_flag = jnp.maximum(end_flag, (ntok == float(BLANK)).astype(F32))
            cur_tok = jnp.where(end_flag > 0, plast, ntok)
            dead = jnp.maximum(end_flag, (enc_len <= i_f).astype(F32))
            hyp = jnp.where(dead > 0, float(BLANK), ntok)
            s1 = jax.lax.convert_element_type(3 * i + (j + 1), F32)
            hyp_ref[...] = jnp.where(lane_h == s1, hyp, phyp)
            last_tok = jnp.where(hyp != 0, hyp, plast)
            perm2 = (iota40_80 == bh40 + 40.0 * end_flag).astype(F32)
            st_ref[...] = _dot(perm2,
                               jnp.concatenate([ns, st_ref[...]], axis=0))
        return cur_tok, last_tok, scores

    zeros40 = jnp.zeros((BB, 1), F32)
    _, _, scores = jax.lax.fori_loop(
        0, T, step_i, (zeros40, zeros40, sinit_ref[...]))

    m = jnp.max(scores, axis=1, keepdims=True)
    bidx = jnp.min(jnp.where(scores == m, iota5_8, 1.0e9), axis=1,
                   keepdims=True)
    oh = (ri8_40 == 5.0 * bi8_40 + bidx).astype(F32)
    out_h_ref[...] = _dot(oh, hyp_ref[...]).astype(jnp.int32)
    out_s_ref[...] = m


def kernel(encoder_outputs, encoder_lengths, emb_table, W_dec, U_dec, b_dec,
           W_enc_j, W_dec_j, b_j, W_out, b_out):
    enc_flat = jnp.transpose(encoder_outputs, (1, 0, 2)).reshape(T * BATCH, D)
    enc_len40 = jnp.repeat(encoder_lengths.astype(F32), BEAM)[:, None]
    scores_init = jnp.tile(
        jnp.asarray([[0.0] + [NEG_BIG] * (BEAM - 1)], F32), (BATCH, 1))
    t1 = emb_table.astype(jnp.bfloat16)
    r1 = emb_table - t1.astype(F32)
    t2 = r1.astype(jnp.bfloat16)
    t3 = (r1 - t2.astype(F32)).astype(jnp.bfloat16)
    t123 = jnp.concatenate([t1, t2, t3], axis=1)
    out_h, out_s = pl.pallas_call(
        _body,
        out_shape=(
            jax.ShapeDtypeStruct((BATCH, HYP_W), jnp.int32),
            jax.ShapeDtypeStruct((BATCH, 1), F32),
        ),
        scratch_shapes=[
            pltpu.VMEM((T * BATCH, D), F32),
            pltpu.VMEM((BB, D), F32),
            pltpu.VMEM((BB, HYP_W), F32),
        ],
        compiler_params=pltpu.CompilerParams(
            vmem_limit_bytes=100 * 1024 * 1024),
    )(enc_flat, W_enc_j, t123, W_dec, U_dec, b_dec.reshape(1, D),
      W_dec_j, b_j.reshape(1, D), W_out, b_out.reshape(1, VOCAB),
      enc_len40, scores_init)
    return out_h[:, : 1 + T * MAX_SYM], out_s[:, 0]


# speedup vs baseline: 1.0586x; 1.0586x over previous
import jax
import jax.numpy as jnp
from jax.experimental import pallas as pl
from jax.experimental.pallas import tpu as pltpu

BATCH = 8
T = 32
D = 512
VOCAB = 4096
BEAM = 5
BB = BATCH * BEAM
MAX_SYM = 3
BLANK = 0
HYP_W = 128
NEG_BIG = -1.0e30
NEG_SENT = -3.0e38

F32 = jnp.float32
HI = jax.lax.Precision.HIGHEST


def _iota(shape, dim):
    return jax.lax.broadcasted_iota(jnp.int32, shape, dim).astype(F32)


def _dot(a, b):
    return jax.lax.dot(a, b, precision=HI, preferred_element_type=F32)


def _ddot(a, b):
    return jax.lax.dot(a, b, preferred_element_type=F32)


def _topk5_iter(work, iot):
    vals, idxs = [], []
    for _ in range(5):
        m = jnp.max(work, axis=1, keepdims=True)
        hit = work == m
        ix = jnp.min(jnp.where(hit, iot, 1.0e9), axis=1, keepdims=True)
        vals.append(m)
        idxs.append(ix)
        work = jnp.where(iot == ix, NEG_SENT, work)
    return jnp.concatenate(vals, axis=1), jnp.concatenate(idxs, axis=1)


def _roll1(x, s):
    return jnp.concatenate([x[:, s:], x[:, :s]], axis=1)


def _topk5_rank(v, ix):
    n = v.shape[1]
    rank = jnp.zeros_like(v)
    for s in range(1, n):
        vr = _roll1(v, s)
        ir = _roll1(ix, s)
        rank = rank + ((vr > v) | ((vr == v) & (ir < ix))).astype(F32)
    vals, idxs = [], []
    for k in range(5):
        hit = rank == float(k)
        vals.append(jnp.sum(jnp.where(hit, v, 0.0), axis=1, keepdims=True))
        idxs.append(jnp.sum(jnp.where(hit, ix, 0.0), axis=1, keepdims=True))
    return jnp.concatenate(vals, axis=1), jnp.concatenate(idxs, axis=1)


def _topk5_wide(work, iot, chunks):
    cw = work.shape[1] // chunks
    cvs, cis = [], []
    for c in range(chunks):
        cv_, ci_ = _topk5_iter(work[:, c * cw:(c + 1) * cw],
                               iot[:, c * cw:(c + 1) * cw])
        cvs.append(cv_)
        cis.append(ci_)
    return _topk5_rank(jnp.concatenate(cvs, axis=1),
                       jnp.concatenate(cis, axis=1))


def _body(enc_ref, wej_ref, t123_ref, wdec_ref, udec_ref, bdec_ref,
          wdj_ref, bj_ref, wout_ref, bout_ref, len_ref, sinit_ref,
          out_h_ref, out_s_ref, encp_ref, st_ref, hyp_ref):
    ri40_8 = _iota((BB, BATCH), 0)
    bi40_8 = _iota((BB, BATCH), 1)
    rep = ((ri40_8 >= 5.0 * bi40_8) & (ri40_8 < 5.0 * bi40_8 + 5.0)).astype(F32)
    gkT = [(ri40_8 == 5.0 * bi40_8 + k).astype(F32) for k in range(BEAM)]
    bi8_40 = _iota((BATCH, BB), 0)
    ri8_40 = _iota((BATCH, BB), 1)
    gp = [(ri8_40 == 5.0 * bi8_40 + p).astype(F32) for p in range(BEAM)]
    base540 = _dot(rep, 5.0 * _iota((BATCH, 1), 0))
    lane_h = _iota((BB, HYP_W), 1)
    iota_v = _iota((BB, VOCAB), 1)
    iota25 = _iota((BATCH, BEAM * BEAM), 1)
    iota5_40 = _iota((BB, BEAM), 1)
    iota5_8 = _iota((BATCH, BEAM), 1)
    iota40_40 = _iota((BB, BB), 1)
    iota40_80 = _iota((BB, 2 * BB), 1)
    pi5_25 = _iota((BEAM, BEAM * BEAM), 0)
    ci5_25 = _iota((BEAM, BEAM * BEAM), 1)
    r5 = ((ci5_25 >= 5.0 * pi5_25) &
          (ci5_25 < 5.0 * pi5_25 + 5.0)).astype(F32)

    def cv(x):
        acc = jnp.zeros((BB, 1), F32)
        for k in range(BEAM):
            acc = acc + _dot(gkT[k], x[:, k:k + 1])
        return acc

    encp_ref[...] = _ddot(enc_ref[...], wej_ref[...])
    st_ref[...] = jnp.zeros((BB, D), F32)
    hyp_ref[...] = jnp.zeros((BB, HYP_W), F32)
    bdec = bdec_ref[...]
    bj = bj_ref[...]
    bout = bout_ref[...]
    enc_len = len_ref[...]

    def step_i(i, carry):
        cur_tok, last_tok, scores = carry
        i_f = jax.lax.convert_element_type(i, F32)
        enc8 = encp_ref[pl.ds(i * BATCH, BATCH), :]
        enc40p = _dot(rep, enc8)
        end_flag = jnp.zeros((BB, 1), F32)
        for j in range(MAX_SYM):
            flag = jnp.maximum(end_flag, (enc_len <= i_f).astype(F32))

            def live_path(tok):
                ohb = (iota_v == tok).astype(jnp.bfloat16)
                e123 = jax.lax.dot(ohb, t123_ref[...],
                                   preferred_element_type=F32)
                emb = ((e123[:, :D] + e123[:, D:2 * D]) + e123[:, 2 * D:])
                ns = jnp.tanh(_ddot(emb, wdec_ref[...]) +
                              _ddot(st_ref[...], udec_ref[...]) + bdec)
                h = jnp.tanh(enc40p + _ddot(ns, wdj_ref[...]) + bj)
                logits = _ddot(h, wout_ref[...]) + bout
                mx = jnp.max(logits, axis=1, keepdims=True)
                shifted = logits - mx
                ls = shifted - jnp.log(jnp.sum(jnp.exp(shifted), axis=1,
                                               keepdims=True))
                tv, ti = _topk5_wide(ls, iota_v, chunks=4)
                return tv, ti, ns

            def dead_path(tok):
                zero5 = jnp.zeros((BB, BEAM), F32)
                return zero5, zero5, jnp.zeros((BB, D), F32)

            tv, ti, ns = jax.lax.cond(jnp.min(flag) >= 1.0,
                                      dead_path, live_path, cur_tok)
            tv = jnp.where((iota5_40 > 0) & (flag > 0), NEG_BIG, tv)
            tv = jnp.where((iota5_40 == 0) & (flag > 0), 0.0, tv)
            ti = jnp.where(flag > 0, float(BLANK), ti)
            tvb = jnp.concatenate([_dot(gp[p], tv) for p in range(BEAM)],
                                  axis=1)
            a = _dot(scores, r5) + tvb
            tsv, tsi = _topk5_rank(a, iota25)
            scores = tsv
            tsi40 = cv(tsi)
            p40 = ((tsi40 >= 5.0).astype(F32) + (tsi40 >= 10.0).astype(F32) +
                   (tsi40 >= 15.0).astype(F32) + (tsi40 >= 20.0).astype(F32))
            q40 = tsi40 - 5.0 * p40
            bh40 = base540 + p40
            perm = (iota40_40 == bh40).astype(F32)
            pt = _dot(perm, jnp.concatenate([hyp_ref[...], ti, last_tok],
                                            axis=1))
            phyp = pt[:, :HYP_W]
            ptop = pt[:, HYP_W:HYP_W + BEAM]
            plast = pt[:, HYP_W + BEAM:HYP_W + BEAM + 1]
            ntok = jnp.sum(ptop * (iota5_40 == q40).astype(F32), axis=1,
                           keepdims=True)
            end_flag = jnp.maximum(end_flag, (ntok == float(BLANK)).astype(F32))
            cur_tok = jnp.where(end_flag > 0, plast, ntok)
            dead = jnp.maximum(end_flag, (enc_len <= i_f).astype(F32))
            hyp = jnp.where(dead > 0, float(BLANK), ntok)
            s1 = jax.lax.convert_element_type(3 * i + (j + 1), F32)
            hyp_ref[...] = jnp.where(lane_h == s1, hyp, phyp)
            last_tok = jnp.where(hyp != 0, hyp, plast)
            perm2 = (iota40_80 == bh40 + 40.0 * end_flag).astype(F32)
            st_ref[...] = _dot(perm2,
                               jnp.concatenate([ns, st_ref[...]], axis=0))
        return cur_tok, last_tok, scores

    zeros40 = jnp.zeros((BB, 1), F32)
    _, _, scores = jax.lax.fori_loop(
        0, T, step_i, (zeros40, zeros40, sinit_ref[...]))

    m = jnp.max(scores, axis=1, keepdims=True)
    bidx = jnp.min(jnp.where(scores == m, iota5_8, 1.0e9), axis=1,
                   keepdims=True)
    oh = (ri8_40 == 5.0 * bi8_40 + bidx).astype(F32)
    out_h_ref[...] = _dot(oh, hyp_ref[...]).astype(jnp.int32)
    out_s_ref[...] = m


def kernel(encoder_outputs, encoder_lengths, emb_table, W_dec, U_dec, b_dec,
           W_enc_j, W_dec_j, b_j, W_out, b_out):
    enc_flat = jnp.transpose(encoder_outputs, (1, 0, 2)).reshape(T * BATCH, D)
    enc_len40 = jnp.repeat(encoder_lengths.astype(F32), BEAM)[:, None]
    scores_init = jnp.tile(
        jnp.asarray([[0.0] + [NEG_BIG] * (BEAM - 1)], F32), (BATCH, 1))
    t1 = emb_table.astype(jnp.bfloat16)
    r1 = emb_table - t1.astype(F32)
    t2 = r1.astype(jnp.bfloat16)
    t3 = (r1 - t2.astype(F32)).astype(jnp.bfloat16)
    t123 = jnp.concatenate([t1, t2, t3], axis=1)
    out_h, out_s = pl.pallas_call(
        _body,
        out_shape=(
            jax.ShapeDtypeStruct((BATCH, HYP_W), jnp.int32),
            jax.ShapeDtypeStruct((BATCH, 1), F32),
        ),
        scratch_shapes=[
            pltpu.VMEM((T * BATCH, D), F32),
            pltpu.VMEM((BB, D), F32),
            pltpu.VMEM((BB, HYP_W), F32),
        ],
        compiler_params=pltpu.CompilerParams(
            vmem_limit_bytes=100 * 1024 * 1024),
    )(enc_flat, W_enc_j, t123, W_dec, U_dec, b_dec.reshape(1, D),
      W_dec_j, b_j.reshape(1, D), W_out, b_out.reshape(1, VOCAB),
      enc_len40, scores_init)
    return out_h[:, : 1 + T * MAX_SYM], out_s[:, 0]
